# baseline (speedup 1.0000x reference)
import functools

import jax
import jax.numpy as jnp
from jax import lax
from jax.experimental import pallas as pl
from jax.experimental.pallas import tpu as pltpu
from jax.experimental.pallas import tpu_sc as plsc

N = 10000
NP = 10240
D = 128
E = 320000
CHUNK = 128
NCHUNKS = 80
NBUF = 2
EDGES_PER_TILE = NCHUNKS * CHUNK
EP = 32 * EDGES_PER_TILE
ROWS_PER_TILE = NP // 16
TRASH = N

_MESH = plsc.VectorSubcoreMesh(core_axis_name="c", subcore_axis_name="s")


def _sc_deg_body(dst_hbm, zer_hbm, ones_hbm, out_hbm, onesv, didx, acc, sem):
    c = lax.axis_index("c")
    s = lax.axis_index("s")
    wid = c * 16 + s
    pltpu.sync_copy(ones_hbm, onesv)
    pltpu.sync_copy(dst_hbm.at[wid], didx)
    pltpu.sync_copy(zer_hbm, acc.at[pl.ds(s * ROWS_PER_TILE, ROWS_PER_TILE)])
    plsc.subcore_barrier()

    def chunk(j, carry):
        pltpu.sync_copy(onesv, acc.at[didx.at[j]], add=True)
        return carry

    lax.fori_loop(0, NCHUNKS, chunk, 0)
    plsc.subcore_barrier()
    r0 = s * ROWS_PER_TILE
    pltpu.sync_copy(acc.at[pl.ds(r0, ROWS_PER_TILE)],
                    out_hbm.at[c, pl.ds(r0, ROWS_PER_TILE)])


_sc_deg = functools.partial(
    pl.kernel,
    out_type=jax.ShapeDtypeStruct((2, NP, D), jnp.float32),
    mesh=_MESH,
    scratch_types=[
        pltpu.VMEM((CHUNK, D), jnp.float32),
        pltpu.VMEM((NCHUNKS, CHUNK), jnp.int32),
        pltpu.VMEM_SHARED((NP, D), jnp.float32),
        pltpu.SemaphoreType.DMA,
    ],
)(_sc_deg_body)


def _sc_gs_body(h_hbm, src_hbm, dst_hbm, zer_hbm, out_hbm,
                sidxr, didx, rows, acc, s0, s1):
    c = lax.axis_index("c")
    s = lax.axis_index("s")
    wid = c * 16 + s
    base = wid * NCHUNKS
    sems = (s0, s1)
    pltpu.sync_copy(dst_hbm.at[wid], didx)
    pltpu.sync_copy(zer_hbm, acc.at[pl.ds(s * ROWS_PER_TILE, ROWS_PER_TILE)])
    plsc.subcore_barrier()

    for b in range(NBUF):
        pltpu.sync_copy(src_hbm.at[base + b], sidxr.at[b])
        pltpu.async_copy(h_hbm.at[sidxr.at[b]], rows.at[b], sems[b])

    def group(g, carry):
        for b in range(NBUF):
            j = g * NBUF + b
            pltpu.make_async_copy(h_hbm.at[sidxr.at[b]], rows.at[b],
                                  sems[b]).wait()
            pltpu.sync_copy(rows.at[b], acc.at[didx.at[j]], add=True)
            pltpu.sync_copy(src_hbm.at[base + j + NBUF], sidxr.at[b])
            pltpu.async_copy(h_hbm.at[sidxr.at[b]], rows.at[b], sems[b])
        return carry

    ngroups = NCHUNKS // NBUF
    lax.fori_loop(0, ngroups - 1, group, 0)
    for b in range(NBUF):
        j = (ngroups - 1) * NBUF + b
        pltpu.make_async_copy(h_hbm.at[sidxr.at[b]], rows.at[b],
                              sems[b]).wait()
        pltpu.sync_copy(rows.at[b], acc.at[didx.at[j]], add=True)

    plsc.subcore_barrier()
    r0 = s * ROWS_PER_TILE
    pltpu.sync_copy(acc.at[pl.ds(r0, ROWS_PER_TILE)],
                    out_hbm.at[c, pl.ds(r0, ROWS_PER_TILE)])


_sc_gs = functools.partial(
    pl.kernel,
    out_type=jax.ShapeDtypeStruct((2, NP, D), jnp.float32),
    mesh=_MESH,
    scratch_types=[
        pltpu.VMEM((NBUF, CHUNK), jnp.int32),
        pltpu.VMEM((NCHUNKS, CHUNK), jnp.int32),
        pltpu.VMEM((NBUF, CHUNK, D), jnp.float32),
        pltpu.VMEM_SHARED((NP, D), jnp.float32),
        pltpu.SemaphoreType.DMA,
        pltpu.SemaphoreType.DMA,
    ],
)(_sc_gs_body)


_BLK = 1024
_GRID = NP // _BLK


def _dinv_block(dp_ref):
    deg = dp_ref[0, :, 0:1] + dp_ref[1, :, 0:1] + 1.0
    return lax.rsqrt(deg)


def _tc_h1_body(x_ref, w_ref, dp_ref, o_ref):
    dinv = _dinv_block(dp_ref)
    o_ref[:, :] = dinv * jnp.dot(x_ref[:, :], w_ref[:, :],
                                 preferred_element_type=jnp.float32)


def _tc_h1(xp, w1t, dparts):
    return pl.pallas_call(
        _tc_h1_body,
        grid=(_GRID,),
        in_specs=[
            pl.BlockSpec((_BLK, D), lambda i: (i, 0)),
            pl.BlockSpec((D, D), lambda i: (0, 0)),
            pl.BlockSpec((2, _BLK, D), lambda i: (0, i, 0)),
        ],
        out_specs=pl.BlockSpec((_BLK, D), lambda i: (i, 0)),
        out_shape=jax.ShapeDtypeStruct((NP, D), jnp.float32),
    )(xp, w1t, dparts)


def _tc_mid_body(s_ref, h_ref, dp_ref, b_ref, w_ref, o_ref):
    dinv = _dinv_block(dp_ref)
    z = dinv * (s_ref[0, :, :] + s_ref[1, :, :] + h_ref[:, :]) + b_ref[:, :]
    a = jnp.maximum(z, 0.0)
    o_ref[:, :] = dinv * jnp.dot(a, w_ref[:, :],
                                 preferred_element_type=jnp.float32)


def _tc_mid(s1, h1p, dparts, b1r, w2t):
    return pl.pallas_call(
        _tc_mid_body,
        grid=(_GRID,),
        in_specs=[
            pl.BlockSpec((2, _BLK, D), lambda i: (0, i, 0)),
            pl.BlockSpec((_BLK, D), lambda i: (i, 0)),
            pl.BlockSpec((2, _BLK, D), lambda i: (0, i, 0)),
            pl.BlockSpec((1, D), lambda i: (0, 0)),
            pl.BlockSpec((D, D), lambda i: (0, 0)),
        ],
        out_specs=pl.BlockSpec((_BLK, D), lambda i: (i, 0)),
        out_shape=jax.ShapeDtypeStruct((NP, D), jnp.float32),
    )(s1, h1p, dparts, b1r, w2t)


def _tc_out_body(s_ref, h_ref, dp_ref, b_ref, o_ref):
    dinv = _dinv_block(dp_ref)
    o_ref[:, :] = dinv * (s_ref[0, :, :] + s_ref[1, :, :] + h_ref[:, :]) \
        + b_ref[:, :]


def _tc_out(s2, h2p, dparts, b2r):
    return pl.pallas_call(
        _tc_out_body,
        grid=(_GRID,),
        in_specs=[
            pl.BlockSpec((2, _BLK, D), lambda i: (0, i, 0)),
            pl.BlockSpec((_BLK, D), lambda i: (i, 0)),
            pl.BlockSpec((2, _BLK, D), lambda i: (0, i, 0)),
            pl.BlockSpec((1, D), lambda i: (0, 0)),
        ],
        out_specs=pl.BlockSpec((_BLK, D), lambda i: (i, 0)),
        out_shape=jax.ShapeDtypeStruct((NP, D), jnp.float32),
    )(s2, h2p, dparts, b2r)


def kernel(x, adj, W1, b1, W2, b2):
    src = adj[0].astype(jnp.int32)
    dst = adj[1].astype(jnp.int32)
    pad = EP - E
    srcp = jnp.concatenate([src, jnp.zeros((pad,), jnp.int32)])
    dstp = jnp.concatenate([dst, jnp.full((pad,), TRASH, jnp.int32)])
    srcp = srcp.reshape(32 * NCHUNKS, CHUNK)
    dstp = dstp.reshape(32, NCHUNKS, CHUNK)
    xp = jnp.pad(x, ((0, NP - N), (0, 0)))
    w1t = W1.T
    w2t = W2.T
    b1r = b1.reshape(1, D)
    b2r = b2.reshape(1, D)
    zer = jnp.zeros((ROWS_PER_TILE, D), jnp.float32)
    ones128 = jnp.ones((CHUNK, D), jnp.float32)

    dparts = _sc_deg(dstp, zer, ones128)
    h1p = _tc_h1(xp, w1t, dparts)
    s1 = _sc_gs(h1p, srcp, dstp, zer)
    h2p = _tc_mid(s1, h1p, dparts, b1r, w2t)
    s2 = _sc_gs(h2p, srcp, dstp, zer)
    outp = _tc_out(s2, h2p, dparts, b2r)
    return outp[:N]


# speedup vs baseline: 1.0820x; 1.0820x over previous
import functools

import jax
import jax.numpy as jnp
from jax import lax
from jax.experimental import pallas as pl
from jax.experimental.pallas import tpu as pltpu
from jax.experimental.pallas import tpu_sc as plsc

N = 10000
NP = 10240
D = 128
E = 320000
CHUNK = 128
NCHUNKS = 80
EDGES_PER_TILE = NCHUNKS * CHUNK
EP = 32 * EDGES_PER_TILE
ROWS_PER_TILE = NP // 16
TRASH = N
IB = 4
GB = 2

_MESH = plsc.VectorSubcoreMesh(core_axis_name="c", subcore_axis_name="s")


def _sc_deg_body(dst_hbm, zer_hbm, ones_hbm, out_hbm, onesv, didx, acc, sem):
    c = lax.axis_index("c")
    s = lax.axis_index("s")
    wid = c * 16 + s
    pltpu.sync_copy(ones_hbm, onesv)
    pltpu.sync_copy(dst_hbm.at[wid], didx)
    pltpu.sync_copy(zer_hbm, acc.at[pl.ds(s * ROWS_PER_TILE, ROWS_PER_TILE)])
    plsc.subcore_barrier()

    def chunk(j, carry):
        pltpu.sync_copy(onesv, acc.at[didx.at[j]], add=True)
        return carry

    lax.fori_loop(0, NCHUNKS, chunk, 0)
    plsc.subcore_barrier()
    r0 = s * ROWS_PER_TILE
    pltpu.sync_copy(acc.at[pl.ds(r0, ROWS_PER_TILE)],
                    out_hbm.at[c, pl.ds(r0, ROWS_PER_TILE)])


_sc_deg = functools.partial(
    pl.kernel,
    out_type=jax.ShapeDtypeStruct((2, NP, D), jnp.float32),
    mesh=_MESH,
    scratch_types=[
        pltpu.VMEM((CHUNK, D), jnp.float32),
        pltpu.VMEM((NCHUNKS, CHUNK), jnp.int32),
        pltpu.VMEM_SHARED((NP, D), jnp.float32),
        pltpu.SemaphoreType.DMA,
    ],
)(_sc_deg_body)


def _sc_gs_body(h_hbm, idx_hbm, zer_hbm, out_hbm,
                idxb, rows, acc, i0, i1, i2, i3, g0, g1):
    c = lax.axis_index("c")
    s = lax.axis_index("s")
    wid = c * 16 + s
    isems = (i0, i1, i2, i3)
    gsems = (g0, g1)
    pltpu.sync_copy(zer_hbm, acc.at[pl.ds(s * ROWS_PER_TILE, ROWS_PER_TILE)])

    pltpu.sync_copy(idx_hbm.at[wid, 0], idxb.at[0])
    for b in range(1, IB):
        pltpu.async_copy(idx_hbm.at[wid, b], idxb.at[b], isems[b])
    plsc.subcore_barrier()
    pltpu.async_copy(h_hbm.at[idxb.at[0, 0]], rows.at[0], g0)
    pltpu.make_async_copy(idx_hbm.at[wid, 1], idxb.at[1], i1).wait()
    pltpu.async_copy(h_hbm.at[idxb.at[1, 0]], rows.at[1], g1)

    def rnd(r, carry):
        for b in range(IB):
            j = r * IB + b
            gb = b % GB
            pltpu.make_async_copy(h_hbm.at[idxb.at[b, 0]], rows.at[gb],
                                  gsems[gb]).wait()
            pltpu.sync_copy(rows.at[gb], acc.at[idxb.at[b, 1]], add=True)
            pltpu.async_copy(idx_hbm.at[wid, j + IB], idxb.at[b], isems[b])
            bn = (b + GB) % IB
            pltpu.make_async_copy(idx_hbm.at[wid, j + GB], idxb.at[bn],
                                  isems[bn]).wait()
            pltpu.async_copy(h_hbm.at[idxb.at[bn, 0]], rows.at[gb], gsems[gb])
        return carry

    lax.fori_loop(0, (NCHUNKS - IB) // IB, rnd, 0)

    base = NCHUNKS - IB
    for b in range(IB):
        j = base + b
        gb = b % GB
        pltpu.make_async_copy(h_hbm.at[idxb.at[b, 0]], rows.at[gb],
                              gsems[gb]).wait()
        pltpu.sync_copy(rows.at[gb], acc.at[idxb.at[b, 1]], add=True)
        if b + GB < IB:
            bn = b + GB
            pltpu.make_async_copy(idx_hbm.at[wid, j + GB], idxb.at[bn],
                                  isems[bn]).wait()
            pltpu.async_copy(h_hbm.at[idxb.at[bn, 0]], rows.at[gb], gsems[gb])

    plsc.subcore_barrier()
    r0 = s * ROWS_PER_TILE
    pltpu.sync_copy(acc.at[pl.ds(r0, ROWS_PER_TILE)],
                    out_hbm.at[c, pl.ds(r0, ROWS_PER_TILE)])


_sc_gs = functools.partial(
    pl.kernel,
    out_type=jax.ShapeDtypeStruct((2, NP, D), jnp.float32),
    mesh=_MESH,
    scratch_types=[
        pltpu.VMEM((IB, 2, CHUNK), jnp.int32),
        pltpu.VMEM((GB, CHUNK, D), jnp.float32),
        pltpu.VMEM_SHARED((NP, D), jnp.float32),
        pltpu.SemaphoreType.DMA,
        pltpu.SemaphoreType.DMA,
        pltpu.SemaphoreType.DMA,
        pltpu.SemaphoreType.DMA,
        pltpu.SemaphoreType.DMA,
        pltpu.SemaphoreType.DMA,
    ],
)(_sc_gs_body)


_BLK = 1024
_GRID = NP // _BLK


def _dinv_block(dp_ref):
    deg = dp_ref[0, :, 0:1] + dp_ref[1, :, 0:1] + 1.0
    return lax.rsqrt(deg)


def _tc_h1_body(x_ref, w_ref, dp_ref, o_ref):
    dinv = _dinv_block(dp_ref)
    o_ref[:, :] = dinv * jnp.dot(x_ref[:, :], w_ref[:, :],
                                 preferred_element_type=jnp.float32)


def _tc_h1(xp, w1t, dparts):
    return pl.pallas_call(
        _tc_h1_body,
        grid=(_GRID,),
        in_specs=[
            pl.BlockSpec((_BLK, D), lambda i: (i, 0)),
            pl.BlockSpec((D, D), lambda i: (0, 0)),
            pl.BlockSpec((2, _BLK, D), lambda i: (0, i, 0)),
        ],
        out_specs=pl.BlockSpec((_BLK, D), lambda i: (i, 0)),
        out_shape=jax.ShapeDtypeStruct((NP, D), jnp.float32),
    )(xp, w1t, dparts)


def _tc_mid_body(s_ref, h_ref, dp_ref, b_ref, w_ref, o_ref):
    dinv = _dinv_block(dp_ref)
    z = dinv * (s_ref[0, :, :] + s_ref[1, :, :] + h_ref[:, :]) + b_ref[:, :]
    a = jnp.maximum(z, 0.0)
    o_ref[:, :] = dinv * jnp.dot(a, w_ref[:, :],
                                 preferred_element_type=jnp.float32)


def _tc_mid(s1, h1p, dparts, b1r, w2t):
    return pl.pallas_call(
        _tc_mid_body,
        grid=(_GRID,),
        in_specs=[
            pl.BlockSpec((2, _BLK, D), lambda i: (0, i, 0)),
            pl.BlockSpec((_BLK, D), lambda i: (i, 0)),
            pl.BlockSpec((2, _BLK, D), lambda i: (0, i, 0)),
            pl.BlockSpec((1, D), lambda i: (0, 0)),
            pl.BlockSpec((D, D), lambda i: (0, 0)),
        ],
        out_specs=pl.BlockSpec((_BLK, D), lambda i: (i, 0)),
        out_shape=jax.ShapeDtypeStruct((NP, D), jnp.float32),
    )(s1, h1p, dparts, b1r, w2t)


def _tc_out_body(s_ref, h_ref, dp_ref, b_ref, o_ref):
    dinv = _dinv_block(dp_ref)
    o_ref[:, :] = dinv * (s_ref[0, :, :] + s_ref[1, :, :] + h_ref[:, :]) \
        + b_ref[:, :]


def _tc_out(s2, h2p, dparts, b2r):
    return pl.pallas_call(
        _tc_out_body,
        grid=(_GRID,),
        in_specs=[
            pl.BlockSpec((2, _BLK, D), lambda i: (0, i, 0)),
            pl.BlockSpec((_BLK, D), lambda i: (i, 0)),
            pl.BlockSpec((2, _BLK, D), lambda i: (0, i, 0)),
            pl.BlockSpec((1, D), lambda i: (0, 0)),
        ],
        out_specs=pl.BlockSpec((_BLK, D), lambda i: (i, 0)),
        out_shape=jax.ShapeDtypeStruct((NP, D), jnp.float32),
    )(s2, h2p, dparts, b2r)


def kernel(x, adj, W1, b1, W2, b2):
    src = adj[0].astype(jnp.int32)
    dst = adj[1].astype(jnp.int32)
    pad = EP - E
    srcp = jnp.concatenate([src, jnp.zeros((pad,), jnp.int32)])
    dstp = jnp.concatenate([dst, jnp.full((pad,), TRASH, jnp.int32)])
    srcp = srcp.reshape(32, NCHUNKS, CHUNK)
    dstp = dstp.reshape(32, NCHUNKS, CHUNK)
    idx2 = jnp.stack([srcp, dstp], axis=2)
    xp = jnp.pad(x, ((0, NP - N), (0, 0)))
    w1t = W1.T
    w2t = W2.T
    b1r = b1.reshape(1, D)
    b2r = b2.reshape(1, D)
    zer = jnp.zeros((ROWS_PER_TILE, D), jnp.float32)
    ones128 = jnp.ones((CHUNK, D), jnp.float32)

    dparts = _sc_deg(dstp, zer, ones128)
    h1p = _tc_h1(xp, w1t, dparts)
    s1 = _sc_gs(h1p, idx2, zer)
    h2p = _tc_mid(s1, h1p, dparts, b1r, w2t)
    s2 = _sc_gs(h2p, idx2, zer)
    outp = _tc_out(s2, h2p, dparts, b2r)
    return outp[:N]
